# Initial kernel scaffold; baseline (speedup 1.0000x reference)
import jax
import jax.numpy as jnp
from jax.experimental import pallas as pl


def kernel(x, edge_index, W1, b1, W2, b2, W3, b3):
    raise NotImplementedError("write your pallas kernel here")


# speedup vs baseline: 7.1342x; 7.1342x over previous
import jax
import jax.numpy as jnp
from jax import lax
from jax.experimental import pallas as pl
from jax.experimental.pallas import tpu as pltpu
from jax.experimental.pallas import tpu_sc as plsc

N = 10000
D = 128
NC = 2
NS = 16
NW = NC * NS
LANES = 16

CHUNK = 128
CPT = 80
STG = 8
NSTG = CPT // STG
EPT = CPT * CHUNK
EPAD = NW * EPT
NPAD = 10240
RPT = NPAD // NS
RCH = RPT // CHUNK
BLK = 128
GRID = NPAD // BLK

_mesh = plsc.VectorSubcoreMesh(
    core_axis_name="c", subcore_axis_name="s", num_cores=NC, num_subcores=NS
)


def _deg_body(dst_hbm, out_hbm, dstv, cnt):
    c = lax.axis_index("c")
    s = lax.axis_index("s")
    wid = s * NC + c
    pltpu.sync_copy(dst_hbm.at[wid], dstv)

    zv = jnp.zeros((LANES,), jnp.float32)
    ones = jnp.ones((LANES,), jnp.float32)

    @pl.loop(0, NPAD // LANES)
    def _zero(i):
        cnt[pl.ds(i * LANES, LANES)] = zv

    @pl.loop(0, EPT // LANES)
    def _count(k):
        idx = dstv[pl.ds(k * LANES, LANES)]
        plsc.addupdate_scatter(cnt, [idx], ones)

    pltpu.sync_copy(cnt, out_hbm.at[wid])


_sc_params = pltpu.CompilerParams(needs_layout_passes=False)

_deg_call = pl.kernel(
    _deg_body,
    out_type=jax.ShapeDtypeStruct((NW, NPAD), jnp.float32),
    mesh=_mesh,
    compiler_params=_sc_params,
    scratch_types=[
        pltpu.VMEM((EPT,), jnp.int32),
        pltpu.VMEM((NPAD,), jnp.float32),
    ],
)


def _agg_body(table_hbm, src_hbm, dst_hbm, out_hbm,
              sidx, didx, rows, agg, sem0, sem1):
    c = lax.axis_index("c")
    s = lax.axis_index("s")
    wid = s * NC + c

    zv = jnp.zeros((LANES,), jnp.float32)

    @pl.loop(0, CHUNK)
    def _zrow(i):
        for col in range(D // LANES):
            rows[0, i, pl.ds(col * LANES, LANES)] = zv

    for t in range(RCH):
        pltpu.sync_copy(rows.at[0], agg.at[pl.ds(s * RPT + t * CHUNK, CHUNK)])

    plsc.subcore_barrier()

    sems = [sem0, sem1]

    @pl.loop(0, NSTG)
    def _stage(st):
        pltpu.sync_copy(src_hbm.at[wid, pl.ds(st * STG, STG)], sidx)
        pltpu.sync_copy(dst_hbm.at[wid, pl.ds(st * STG, STG)], didx)
        pltpu.async_copy(table_hbm.at[sidx.at[0]], rows.at[0], sems[0])
        for cc in range(STG):
            if cc + 1 < STG:
                pltpu.async_copy(table_hbm.at[sidx.at[cc + 1]],
                                 rows.at[(cc + 1) % 2], sems[(cc + 1) % 2])
            pltpu.make_async_copy(table_hbm.at[sidx.at[cc]],
                                  rows.at[cc % 2], sems[cc % 2]).wait()
            pltpu.sync_copy(rows.at[cc % 2], agg.at[didx.at[cc]], add=True)

    plsc.subcore_barrier()
    pltpu.sync_copy(agg.at[pl.ds(s * RPT, RPT)],
                    out_hbm.at[c, pl.ds(s * RPT, RPT)])


_agg_call = pl.kernel(
    _agg_body,
    out_type=jax.ShapeDtypeStruct((NC, NPAD, D), jnp.float32),
    mesh=_mesh,
    compiler_params=_sc_params,
    scratch_types=[
        pltpu.VMEM((STG, CHUNK), jnp.int32),
        pltpu.VMEM((STG, CHUNK), jnp.int32),
        pltpu.VMEM((2, CHUNK, D), jnp.float32),
        pltpu.VMEM_SHARED((NPAD, D), jnp.float32),
        pltpu.SemaphoreType.DMA,
        pltpu.SemaphoreType.DMA,
    ],
)


def _dis_block(cnt_ref):
    deg = 1.0 + jnp.sum(cnt_ref[...], axis=0)
    return lax.rsqrt(deg)


def _pre_body(cnt_ref, x_ref, w_ref, hp_ref):
    dis = _dis_block(cnt_ref)
    h = jnp.dot(x_ref[...], w_ref[...], preferred_element_type=jnp.float32)
    hp_ref[...] = dis[:, None] * h


def _mid_body(cnt_ref, p_ref, hp_ref, b_ref, w_ref, hpn_ref):
    dis = _dis_block(cnt_ref)
    tot = p_ref[0] + p_ref[1] + hp_ref[...]
    z = jnp.maximum(dis[:, None] * tot + b_ref[...], 0.0)
    h = jnp.dot(z, w_ref[...], preferred_element_type=jnp.float32)
    hpn_ref[...] = dis[:, None] * h


def _fin_body(cnt_ref, p_ref, hp_ref, b_ref, out_ref):
    dis = _dis_block(cnt_ref)
    o = dis[:, None] * (p_ref[0] + p_ref[1] + hp_ref[...]) + b_ref[...]
    m = jnp.max(o, axis=1, keepdims=True)
    e = jnp.exp(o - m)
    lse = jnp.log(jnp.sum(e, axis=1, keepdims=True))
    out_ref[...] = o - m - lse


_cnt_spec = pl.BlockSpec((NW, BLK), lambda i: (0, i))
_row_spec = pl.BlockSpec((BLK, D), lambda i: (i, 0))
_w_spec = pl.BlockSpec((D, D), lambda i: (0, 0))
_b_spec = pl.BlockSpec((1, D), lambda i: (0, 0))
_p_spec = pl.BlockSpec((NC, BLK, D), lambda i: (0, i, 0))
_out_struct = jax.ShapeDtypeStruct((NPAD, D), jnp.float32)

_pre_call = pl.pallas_call(
    _pre_body, grid=(GRID,),
    in_specs=[_cnt_spec, _row_spec, _w_spec],
    out_specs=_row_spec, out_shape=_out_struct,
)
_mid_call = pl.pallas_call(
    _mid_body, grid=(GRID,),
    in_specs=[_cnt_spec, _p_spec, _row_spec, _b_spec, _w_spec],
    out_specs=_row_spec, out_shape=_out_struct,
)
_fin_call = pl.pallas_call(
    _fin_body, grid=(GRID,),
    in_specs=[_cnt_spec, _p_spec, _row_spec, _b_spec],
    out_specs=_row_spec, out_shape=_out_struct,
)


def kernel(x, edge_index, W1, b1, W2, b2, W3, b3):
    src = edge_index[0]
    dst = edge_index[1]
    e = src.shape[0]
    pad = EPAD - e
    src_p = jnp.concatenate(
        [src, jnp.full((pad,), N, jnp.int32)]).reshape(NW, CPT, CHUNK)
    dst_p = jnp.concatenate(
        [dst, jnp.full((pad,), N, jnp.int32)]).reshape(NW, CPT, CHUNK)
    xp = jnp.zeros((NPAD, D), x.dtype).at[:N].set(x)

    counts = _deg_call(dst_p.reshape(NW, EPT))

    b1r = b1.reshape(1, D)
    b2r = b2.reshape(1, D)
    b3r = b3.reshape(1, D)

    hp1 = _pre_call(counts, xp, W1)
    p1 = _agg_call(hp1, src_p, dst_p)
    hp2 = _mid_call(counts, p1, hp1, b1r, W2)
    p2 = _agg_call(hp2, src_p, dst_p)
    hp3 = _mid_call(counts, p2, hp2, b2r, W3)
    p3 = _agg_call(hp3, src_p, dst_p)
    out = _fin_call(counts, p3, hp3, b3r)
    return out[:N]
